# baseline (speedup 1.0000x reference)
import functools
import math

import jax
import jax.numpy as jnp
from jax import lax
from jax.experimental import pallas as pl
from jax.experimental.pallas import tpu as pltpu
from jax.experimental.pallas import tpu_sc as plsc

_D = 768
_SEQ = 2048
_BATCH = 4
_TOTAL = _BATCH * _SEQ
_NC, _NS = 2, 16
_NW = _NC * _NS
_B_PER_W = _TOTAL // _NW
_S_PER_W = _SEQ // _NW
_K = 16
_CPB = _S_PER_W // _K
_NCHUNK = _B_PER_W // _K
_LANES = 16
_VPR = _D // _LANES
_SCALE = math.sqrt(float(_D))
_NB_G = 2
_NB_A = 4

_mesh = plsc.VectorSubcoreMesh(
    core_axis_name="c", subcore_axis_name="s", num_cores=_NC, num_subcores=_NS
)


@functools.partial(
    pl.kernel,
    out_type=jax.ShapeDtypeStruct((_TOTAL, _D), jnp.float32),
    mesh=_mesh,
    scratch_types=[
        pltpu.VMEM((_B_PER_W,), jnp.int32),
        pltpu.VMEM((_S_PER_W, _D), jnp.float32),
        [pltpu.VMEM((_K, _D), jnp.float32) for _ in range(_NB_G)],
        [pltpu.VMEM((_K, _D), jnp.float32) for _ in range(_NB_A)],
        [pltpu.SemaphoreType.DMA for _ in range(_NB_G)],
        [pltpu.SemaphoreType.DMA for _ in range(_NB_A)],
    ],
)
def _embed(
    x_hbm, pos_hbm, table_hbm, out_hbm,
    idx_v, pos_res, gbufs, abufs, gsems, osems,
):
    cid = lax.axis_index("c")
    sid = lax.axis_index("s")
    wid = cid * _NS + sid
    p0 = wid * _S_PER_W

    pltpu.sync_copy(pos_hbm.at[pl.ds(p0, _S_PER_W)], pos_res)
    for b in range(_BATCH):
        pltpu.sync_copy(
            x_hbm.at[pl.ds(b * _SEQ + p0, _S_PER_W)],
            idx_v.at[pl.ds(b * _S_PER_W, _S_PER_W)],
        )

    gathers = [None] * _NB_G
    outs = [None] * _NB_A

    def issue_gather(c):
        g = c % _NB_G
        gathers[g] = pltpu.async_copy(
            table_hbm.at[idx_v.at[pl.ds(c * _K, _K)]], gbufs[g], gsems[g]
        )

    issue_gather(0)
    issue_gather(1)

    for c in range(_NCHUNK):
        g = c % _NB_G
        a = c % _NB_A
        batch = c // _CPB
        off = (c % _CPB) * _K

        if outs[a] is not None:
            outs[a].wait()
            outs[a] = None
        gathers[g].wait()

        gbuf = gbufs[g]
        abuf = abufs[a]

        @plsc.parallel_loop(0, _K, unroll=1)
        def _(r):
            for j in range(_VPR):
                sl = pl.ds(j * _LANES, _LANES)
                abuf[r, sl] = gbuf[r, sl] * _SCALE + pos_res[off + r, sl]

        outs[a] = pltpu.async_copy(
            abuf,
            out_hbm.at[pl.ds(batch * _SEQ + p0 + off, _K)],
            osems[a],
        )
        if c + _NB_G < _NCHUNK:
            issue_gather(c + _NB_G)

    for o in outs:
        if o is not None:
            o.wait()


def kernel(x, token_table, pos_embedding):
    x_flat = x.reshape(_TOTAL).astype(jnp.int32)
    pos2d = pos_embedding.reshape(_SEQ, _D)
    out = _embed(x_flat, pos2d, token_table)
    return out.reshape(_BATCH, _SEQ, _D)
